# baseline (speedup 1.0000x reference)
import jax
import jax.numpy as jnp
from jax.experimental import pallas as pl

DEPTH = 1000
N_ROWS = 4096 * 26
ROWS_PER_BLOCK = 512
N_BLOCKS = N_ROWS // ROWS_PER_BLOCK


def _onehot_block(idx_ref, out_ref):
    idx = idx_ref[0]
    iota = jax.lax.broadcasted_iota(jnp.int32, out_ref.shape, 1)
    out_ref[...] = jnp.where(idx == iota, 1.0, 0.0).astype(jnp.float32)


def kernel(inputs):
    idx3 = inputs.reshape(N_BLOCKS, ROWS_PER_BLOCK, 1)
    out = pl.pallas_call(
        _onehot_block,
        grid=(N_BLOCKS,),
        in_specs=[pl.BlockSpec((1, ROWS_PER_BLOCK, 1), lambda i: (i, 0, 0))],
        out_specs=pl.BlockSpec((ROWS_PER_BLOCK, DEPTH), lambda i: (i, 0)),
        out_shape=jax.ShapeDtypeStruct((N_ROWS, DEPTH), jnp.float32),
    )(idx3)
    return out.reshape(4096, 26, DEPTH)
